# Initial kernel scaffold; baseline (speedup 1.0000x reference)
import jax
import jax.numpy as jnp
from jax.experimental import pallas as pl


def kernel(x, a2a, W, b):
    raise NotImplementedError("write your pallas kernel here")


# speedup vs baseline: 2.0309x; 2.0309x over previous
import functools

import jax
import jax.numpy as jnp
from jax import lax
from jax.experimental import pallas as pl
from jax.experimental.pallas import tpu as pltpu
from jax.experimental.pallas import tpu_sc as plsc

N, K, D = 10000, 32, 128
NPAD = 10240
NW = 32
NPW = NPAD // NW
NBUF = 2
L = 16
NCH = D // L

TC_BLK = 512


def _linear_body(x_ref, wt_ref, b_ref, h_ref):
    h = jnp.dot(x_ref[...], wt_ref[...], preferred_element_type=jnp.float32)
    h = h + b_ref[...]
    row = lax.broadcasted_iota(jnp.int32, h.shape, 0) + pl.program_id(0) * TC_BLK
    h_ref[...] = jnp.where(row == 0, jnp.float32(-9e15), h)


def _linear(x_pad, Wt, b2):
    return pl.pallas_call(
        _linear_body,
        grid=(NPAD // TC_BLK,),
        in_specs=[
            pl.BlockSpec((TC_BLK, D), lambda i: (i, 0)),
            pl.BlockSpec((D, D), lambda i: (0, 0)),
            pl.BlockSpec((1, D), lambda i: (0, 0)),
        ],
        out_specs=pl.BlockSpec((TC_BLK, D), lambda i: (i, 0)),
        out_shape=jax.ShapeDtypeStruct((NPAD, D), jnp.float32),
    )(x_pad, Wt, b2)


def _sc_body(h_hbm, a2a_hbm, out_hbm, idx_v, hs_v, out_v, rows_v, sem):
    wid = lax.axis_index("s") * 2 + lax.axis_index("c")
    base = wid * NPW

    pltpu.sync_copy(a2a_hbm.at[pl.ds(base * K, NPW * K)], idx_v)
    pltpu.sync_copy(h_hbm.at[pl.ds(base, NPW)], hs_v)

    lane = lax.iota(jnp.int32, L)

    for b in range(NBUF):
        pltpu.make_async_copy(h_hbm.at[idx_v.at[pl.ds(b * K, K)]], rows_v.at[b], sem).start()

    def group(g, carry):
        for b in range(NBUF):
            i = g * NBUF + b
            pltpu.make_async_copy(h_hbm.at[idx_v.at[pl.ds(i * K, K)]], rows_v.at[b], sem).wait()

            hc = [hs_v[i, pl.ds(c * L, L)] for c in range(NCH)]
            acc = hc[0] * hc[0]
            for c in range(1, NCH):
                acc = acc + hc[c] * hc[c]
            s0 = jnp.sum(acc)

            sv0 = jnp.zeros((L,), jnp.float32)
            sv1 = jnp.zeros((L,), jnp.float32)
            for k in range(K):
                a = rows_v[b, k, pl.ds(0, L)] * hc[0]
                for c in range(1, NCH):
                    a = a + rows_v[b, k, pl.ds(c * L, L)] * hc[c]
                sk = jnp.full((L,), jnp.sum(a), jnp.float32)
                if k < L:
                    sv0 = jnp.where(lane == k, sk, sv0)
                else:
                    sv1 = jnp.where(lane == (k - L), sk, sv1)

            @pl.when(i + NBUF < NPW)
            def _():
                pltpu.make_async_copy(
                    h_hbm.at[idx_v.at[pl.ds((i + NBUF) * K, K)]], rows_v.at[b], sem
                ).start()

            s0v = jnp.full((L,), s0, jnp.float32)
            z = jnp.sum(jnp.exp(sv0 - s0v) + jnp.exp(sv1 - s0v))
            wv = jnp.full((L,), 1.0, jnp.float32) / jnp.full((L,), 1.0 + z, jnp.float32)
            node = jnp.full((L,), base + i, jnp.int32)
            wv = jnp.where(node == 0, jnp.float32(0.0), wv)
            for c in range(NCH):
                out_v[i, pl.ds(c * L, L)] = hc[c] * wv
        return carry

    lax.fori_loop(0, NPW // NBUF, group, 0)
    pltpu.sync_copy(out_v, out_hbm.at[pl.ds(base, NPW)])


@functools.cache
def _sc_attend():
    return pl.kernel(
        _sc_body,
        mesh=plsc.VectorSubcoreMesh(core_axis_name="c", subcore_axis_name="s"),
        out_type=jax.ShapeDtypeStruct((NPAD, D), jnp.float32),
        scratch_types=[
            pltpu.VMEM((NPW * K,), jnp.int32),
            pltpu.VMEM((NPW, D), jnp.float32),
            pltpu.VMEM((NPW, D), jnp.float32),
            pltpu.VMEM((NBUF, K, D), jnp.float32),
            pltpu.SemaphoreType.DMA,
        ],
        compiler_params=pltpu.CompilerParams(needs_layout_passes=False),
    )


def kernel(x, a2a, W, b):
    x_pad = jnp.zeros((NPAD, D), jnp.float32).at[:N].set(x)
    a2a_pad = jnp.zeros((NPAD, K), jnp.int32).at[:N].set(a2a).reshape(NPAD * K)
    h = _linear(x_pad, W.T, b[None, :])
    out = _sc_attend()(h, a2a_pad)
    return out[:N]
